# Initial kernel scaffold; baseline (speedup 1.0000x reference)
import jax
import jax.numpy as jnp
from jax.experimental import pallas as pl


def kernel(x, batch):
    raise NotImplementedError("write your pallas kernel here")


# speedup vs baseline: 4.9636x; 4.9636x over previous
import jax
import jax.numpy as jnp
from jax import lax
from jax.experimental import pallas as pl
from jax.experimental.pallas import tpu as pltpu
from jax.experimental.pallas import tpu_sc as plsc

N = 320000
D = 128
S = 10000
NW = 32
SPW = 320
S_PAD = NW * SPW
C = 128
RPW = SPW + 1
CNT_PAD = ((SPW + 15) // 16 + 1) * 16


def _pool_kernel(x_hbm, b_hbm, bounds_hbm, out_hbm,
                 xbuf, bbuf, idxbuf, accl, cnt, bnds, acc_sh, sem):
    cid = lax.axis_index("c")
    sid = lax.axis_index("s")
    w = sid * 2 + cid
    seg_lo = w * SPW
    base = sid * RPW

    pltpu.sync_copy(bounds_hbm, bnds)
    bv0 = bnds[pl.ds(w, 16)]
    lo = bv0[0]
    hi = bv0[1]

    zeros16 = jnp.zeros((16,), jnp.float32)

    def zrow(i, carry):
        for j in range(8):
            accl[i, pl.ds(j * 16, 16)] = zeros16
        return carry
    lax.fori_loop(0, RPW, zrow, 0)
    for j in range(CNT_PAD // 16):
        cnt[pl.ds(j * 16, 16)] = zeros16
    pltpu.sync_copy(accl, acc_sh.at[pl.ds(base, RPW)])

    lo_al = lo & jnp.int32(~7)
    nchunks = (hi - lo_al + C - 1) // C

    def chunk(k, carry):
        r = lo_al + k * C
        r_eff = jnp.minimum(r, N - C)
        r_eff = pl.multiple_of(r_eff, 8)
        pltpu.sync_copy(b_hbm.at[pl.ds(r_eff, C)], bbuf)
        cp = pltpu.async_copy(x_hbm.at[pl.ds(r_eff, C)], xbuf, sem)
        vlo = jnp.maximum(r, lo)
        def lanes(j, c2):
            bv = bbuf[pl.ds(j * 16, 16)]
            g = r_eff + j * 16 + lax.iota(jnp.int32, 16)
            valid = (g >= vlo) & (g < hi)
            loc = jnp.where(valid, bv - seg_lo, SPW)
            idxbuf[pl.ds(j * 16, 16)] = base + loc
            ones = jnp.where(valid, 1.0, 0.0).astype(jnp.float32)
            plsc.addupdate_scatter(cnt, [loc], ones)
            return c2
        lax.fori_loop(0, C // 16, lanes, 0)
        cp.wait()
        pltpu.sync_copy(xbuf, acc_sh.at[idxbuf], add=True)
        return carry
    lax.fori_loop(0, nchunks, chunk, 0)

    pltpu.sync_copy(acc_sh.at[pl.ds(base, SPW)], accl.at[pl.ds(0, SPW)])

    def div_row(s, carry):
        cv = cnt[pl.ds(s, 16)]
        inv = (jnp.ones((16,), jnp.float32) / jnp.maximum(cv, 1.0))[0]
        for j in range(8):
            accl[s, pl.ds(j * 16, 16)] = accl[s, pl.ds(j * 16, 16)] * inv
        return carry
    lax.fori_loop(0, SPW, div_row, 0)

    pltpu.sync_copy(accl.at[pl.ds(0, SPW)], out_hbm.at[pl.ds(seg_lo, SPW)])


def kernel(x, batch):
    b32 = batch.astype(jnp.int32)
    edges = jnp.arange(NW + 1, dtype=jnp.int32) * SPW
    bounds = jnp.searchsorted(b32, edges, side="left").astype(jnp.int32)
    bounds = jnp.concatenate([bounds, jnp.zeros((15,), jnp.int32)])

    mesh = plsc.VectorSubcoreMesh(core_axis_name="c", subcore_axis_name="s")
    out = pl.kernel(
        _pool_kernel,
        mesh=mesh,
        compiler_params=pltpu.CompilerParams(needs_layout_passes=False),
        out_type=jax.ShapeDtypeStruct((S_PAD, D), jnp.float32),
        scratch_types=[
            pltpu.VMEM((C, D), jnp.float32),
            pltpu.VMEM((C,), jnp.int32),
            pltpu.VMEM((C,), jnp.int32),
            pltpu.VMEM((RPW, D), jnp.float32),
            pltpu.VMEM((CNT_PAD,), jnp.float32),
            pltpu.VMEM((48,), jnp.int32),
            pltpu.VMEM_SHARED((16 * RPW, D), jnp.float32),
            pltpu.SemaphoreType.DMA,
        ],
    )(x, b32, bounds)
    return out[:S]


# speedup vs baseline: 7.5209x; 1.5152x over previous
import jax
import jax.numpy as jnp
from jax import lax
from jax.experimental import pallas as pl
from jax.experimental.pallas import tpu as pltpu
from jax.experimental.pallas import tpu_sc as plsc

N = 320000
D = 128
S = 10000
NW = 32
SPW = 320
S_PAD = NW * SPW
C = 128
SUB = 16
BCH = SUB * C
RPW = SPW + 1
CNT_PAD = ((SPW + 15) // 16 + 1) * 16


def _pool_kernel(x_hbm, b_hbm, bounds_hbm, out_hbm,
                 xbuf, bbuf, idxb, accl, cnt, bnds, acc_sh,
                 sx0, sx1, ss0, ss1):
    cid = lax.axis_index("c")
    sid = lax.axis_index("s")
    w = sid * 2 + cid
    seg_lo = w * SPW
    base = sid * RPW

    sx = (sx0, sx1)
    ss = (ss0, ss1)

    pltpu.sync_copy(bounds_hbm, bnds)
    bv0 = bnds[pl.ds(w, 16)]
    lo = bv0[0]
    hi = bv0[1]

    zeros16 = jnp.zeros((16,), jnp.float32)

    def zrow(i, carry):
        for j in range(8):
            accl[i, pl.ds(j * 16, 16)] = zeros16
        return carry
    lax.fori_loop(0, RPW, zrow, 0)
    for j in range(CNT_PAD // 16):
        cnt[pl.ds(j * 16, 16)] = zeros16
    pltpu.sync_copy(accl, acc_sh.at[pl.ds(base, RPW)])

    lo_al = lo & jnp.int32(~7)
    nchunks = (hi - lo_al + C - 1) // C

    def xload(k, b):
        r_eff = pl.multiple_of(jnp.minimum(lo_al + k * C, N - C), 8)
        pltpu.async_copy(x_hbm.at[pl.ds(r_eff, C)], xbuf.at[b], sx[b])

    def xwait(b):
        pltpu.make_async_copy(x_hbm.at[pl.ds(0, C)], xbuf.at[b],
                              sx[b]).wait()

    def scat_wait(b):
        pltpu.make_async_copy(xbuf.at[b], acc_sh.at[idxb.at[b]],
                              ss[b]).wait()

    @pl.when(nchunks > 0)
    def _():
        xload(0, 0)

    def pair(p, carry):
        for b in range(2):
            k = 2 * p + b

            @pl.when(k < nchunks)
            def _():
                s_sup = k // SUB
                rb_eff = pl.multiple_of(
                    jnp.minimum(lo_al + s_sup * BCH, N - BCH), 8)

                @pl.when(lax.rem(k, SUB) == 0)
                def _():
                    pltpu.sync_copy(b_hbm.at[pl.ds(rb_eff, BCH)], bbuf)

                @pl.when(k > 0)
                def _():
                    scat_wait(1 - b)

                @pl.when(k + 1 < nchunks)
                def _():
                    xload(k + 1, 1 - b)

                r = lo_al + k * C
                r_eff = pl.multiple_of(jnp.minimum(r, N - C), 8)
                off = r_eff - rb_eff
                vlo = jnp.maximum(r, lo)
                for j in range(C // 16):
                    bv = bbuf[pl.ds(off + j * 16, 16)]
                    g = r_eff + j * 16 + lax.iota(jnp.int32, 16)
                    valid = (g >= vlo) & (g < hi)
                    loc = jnp.where(valid, bv - seg_lo, SPW)
                    idxb[b, pl.ds(j * 16, 16)] = base + loc
                    ones = jnp.where(valid, 1.0, 0.0).astype(jnp.float32)
                    plsc.addupdate_scatter(cnt, [loc], ones)

                xwait(b)
                pltpu.async_copy(xbuf.at[b], acc_sh.at[idxb.at[b]], ss[b],
                                 add=True)
        return carry
    lax.fori_loop(0, (nchunks + 1) // 2, pair, 0)

    last_even = lax.rem(nchunks - 1, 2) == 0

    @pl.when((nchunks > 0) & last_even)
    def _():
        scat_wait(0)

    @pl.when((nchunks > 0) & jnp.logical_not(last_even))
    def _():
        scat_wait(1)

    pltpu.sync_copy(acc_sh.at[pl.ds(base, SPW)], accl.at[pl.ds(0, SPW)])

    def div_row(s, carry):
        cv = cnt[pl.ds(s, 16)]
        inv = (jnp.ones((16,), jnp.float32) / jnp.maximum(cv, 1.0))[0]
        for j in range(8):
            accl[s, pl.ds(j * 16, 16)] = accl[s, pl.ds(j * 16, 16)] * inv
        return carry
    lax.fori_loop(0, SPW, div_row, 0)

    pltpu.sync_copy(accl.at[pl.ds(0, SPW)], out_hbm.at[pl.ds(seg_lo, SPW)])


def kernel(x, batch):
    b32 = batch.astype(jnp.int32)
    edges = jnp.arange(NW + 1, dtype=jnp.int32) * SPW
    bounds = jnp.searchsorted(b32, edges, side="left").astype(jnp.int32)
    bounds = jnp.concatenate([bounds, jnp.zeros((15,), jnp.int32)])

    mesh = plsc.VectorSubcoreMesh(core_axis_name="c", subcore_axis_name="s")
    out = pl.kernel(
        _pool_kernel,
        mesh=mesh,
        compiler_params=pltpu.CompilerParams(needs_layout_passes=False),
        out_type=jax.ShapeDtypeStruct((S_PAD, D), jnp.float32),
        scratch_types=[
            pltpu.VMEM((2, C, D), jnp.float32),
            pltpu.VMEM((BCH,), jnp.int32),
            pltpu.VMEM((2, C), jnp.int32),
            pltpu.VMEM((RPW, D), jnp.float32),
            pltpu.VMEM((CNT_PAD,), jnp.float32),
            pltpu.VMEM((48,), jnp.int32),
            pltpu.VMEM_SHARED((16 * RPW, D), jnp.float32),
            pltpu.SemaphoreType.DMA,
            pltpu.SemaphoreType.DMA,
            pltpu.SemaphoreType.DMA,
            pltpu.SemaphoreType.DMA,
        ],
    )(x, b32, bounds)
    return out[:S]
